# baseline (speedup 1.0000x reference)
import functools

import jax
import jax.numpy as jnp
from jax import lax
from jax.experimental import pallas as pl
from jax.experimental.pallas import tpu as pltpu
from jax.experimental.pallas import tpu_sc as plsc

_NC = 2
_NS = 16
_L = 16

_CHUNK = 128


@functools.lru_cache(maxsize=None)
def _make_sc_spmm(N, D, R, EPTP, NPAD):
    assert R % _NC == 0 and NPAD % (8 * _NS) == 0 and EPTP % _CHUNK == 0
    rpc = R // _NC
    n_chunks = EPTP // _CHUNK
    rpt = NPAD // _NS
    mesh = plsc.VectorSubcoreMesh(core_axis_name="c", subcore_axis_name="s")

    @functools.partial(
        pl.kernel,
        out_type=jax.ShapeDtypeStruct((R * NPAD, D), jnp.float32),
        mesh=mesh,
        scratch_types=[
            [pltpu.VMEM((_CHUNK,), jnp.int32) for _ in range(2)],
            pltpu.VMEM((_CHUNK,), jnp.int32),
            pltpu.VMEM((_CHUNK,), jnp.float32),
            pltpu.VMEM((_CHUNK, D), jnp.float32),
            pltpu.VMEM_SHARED((NPAD, D), jnp.float32),
            pltpu.SemaphoreType.DMA,
        ],
    )
    def spmm(x_hbm, src_hbm, dst_hbm, w_hbm, out_hbm, src_v, dst_v, w_v,
             rows_v, acc, sem):
        cid = lax.axis_index("c")
        sid = lax.axis_index("s")
        row0 = sid * rpt

        for rr in range(rpc):
            r = cid * rpc + rr

            def _zb(i, carry):
                z = jnp.zeros((_L,), jnp.float32)
                for j in range(D // _L):
                    rows_v[i, pl.ds(j * _L, _L)] = z
                return carry
            lax.fori_loop(0, _CHUNK, _zb, 0)

            done = 0
            while done < rpt:
                nrows = min(_CHUNK, rpt - done)
                pltpu.sync_copy(rows_v.at[pl.ds(0, nrows)],
                                acc.at[pl.ds(row0 + done, nrows)])
                done += nrows
            plsc.subcore_barrier()

            seg = (r * _NS + sid) * n_chunks

            def _scale(c16, c2):
                wvec = w_v[pl.ds(c16 * _L, _L)]
                for e16 in range(_L):
                    w = wvec[e16]
                    e = c16 * _L + e16
                    for j in range(D // _L):
                        sl = pl.ds(j * _L, _L)
                        rows_v[e, sl] = rows_v[e, sl] * w
                return c2

            def _half(k, p):
                desc = pltpu.async_copy(x_hbm.at[src_v[p]], rows_v, sem)
                pltpu.sync_copy(dst_hbm.at[seg + k], dst_v)
                pltpu.sync_copy(w_hbm.at[seg + k], w_v)
                kn = jnp.minimum(k + 1, n_chunks - 1)
                pltpu.sync_copy(src_hbm.at[seg + kn], src_v[1 - p])
                desc.wait()
                lax.fori_loop(0, _CHUNK // _L, _scale, 0)
                pltpu.sync_copy(rows_v, acc.at[dst_v], add=True)

            pltpu.sync_copy(src_hbm.at[seg], src_v[0])

            def _pair(k2, carry):
                _half(k2 * 2, 0)
                _half(k2 * 2 + 1, 1)
                return carry
            lax.fori_loop(0, n_chunks // 2, _pair, 0)
            plsc.subcore_barrier()

            done = 0
            while done < rpt:
                nrows = min(_CHUNK, rpt - done)
                pltpu.sync_copy(acc.at[pl.ds(row0 + done, nrows)],
                                out_hbm.at[pl.ds(r * NPAD + row0 + done,
                                                 nrows)])
                done += nrows

    return spmm


@functools.lru_cache(maxsize=None)
def _make_dense(N, D, R, DOUT, B):
    assert N % B == 0
    grid = (N // B,)

    def body(s_ref, w_ref, sw_ref, b_ref, g_ref, be_ref, o_ref):
        ssum = s_ref[0]
        for r in range(1, R):
            ssum = ssum + s_ref[r]
        mu = jnp.mean(ssum, axis=-1, keepdims=True)
        d = ssum - mu
        var = jnp.mean(d * d, axis=-1, keepdims=True)
        sn = d * lax.rsqrt(var + 1e-6) * g_ref[...] + be_ref[...]
        acc = jnp.dot(sn, sw_ref[...], preferred_element_type=jnp.float32)
        for r in range(R):
            acc = acc + jnp.dot(s_ref[r], w_ref[r],
                                preferred_element_type=jnp.float32)
        o_ref[...] = acc * 0.5 + b_ref[...]

    return pl.pallas_call(
        body,
        grid=grid,
        in_specs=[
            pl.BlockSpec((R, B, D), lambda i: (0, i, 0)),
            pl.BlockSpec((R, D, DOUT), lambda i: (0, 0, 0)),
            pl.BlockSpec((D, DOUT), lambda i: (0, 0)),
            pl.BlockSpec((1, DOUT), lambda i: (0, 0)),
            pl.BlockSpec((1, D), lambda i: (0, 0)),
            pl.BlockSpec((1, D), lambda i: (0, 0)),
        ],
        out_specs=pl.BlockSpec((B, DOUT), lambda i: (i, 0)),
        out_shape=jax.ShapeDtypeStruct((N, DOUT), jnp.float32),
    )


def kernel(x, edge_index, edge_weight, weight, share_weight, bias,
           ln_gamma, ln_beta):
    N, D = x.shape
    R, _, E = edge_index.shape
    DOUT = weight.shape[1]

    ept = E // _NS
    n_chunks = -(-ept // _CHUNK)
    n_chunks += n_chunks % 2
    eptp = n_chunks * _CHUNK
    pad = eptp - ept

    src = edge_index[:, 0, :].reshape(R, _NS, ept)
    dst = edge_index[:, 1, :].reshape(R, _NS, ept)
    ew = edge_weight.reshape(R, _NS, ept)
    if pad:
        src = jnp.pad(src, ((0, 0), (0, 0), (0, pad)))
        dst = jnp.pad(dst, ((0, 0), (0, 0), (0, pad)))
        ew = jnp.pad(ew, ((0, 0), (0, 0), (0, pad)))
    srcc = src.reshape(R * _NS * n_chunks, _CHUNK)
    dstc = dst.reshape(R * _NS * n_chunks, _CHUNK)
    wchunks = ew.reshape(R * _NS * n_chunks, _CHUNK)

    npad = -(-N // (8 * _NS)) * (8 * _NS)
    supports = _make_sc_spmm(N, D, R, eptp, npad)(x, srcc, dstc, wchunks)
    s = supports.reshape(R, npad, D)

    dense = _make_dense(N, D, R, DOUT, B=1000)
    return dense(
        s,
        weight.reshape(R, D, DOUT),
        share_weight,
        bias.reshape(1, DOUT),
        ln_gamma.reshape(1, D),
        ln_beta.reshape(1, D),
    )


# speedup vs baseline: 1.4466x; 1.4466x over previous
import functools

import jax
import jax.numpy as jnp
from jax import lax
from jax.experimental import pallas as pl
from jax.experimental.pallas import tpu as pltpu
from jax.experimental.pallas import tpu_sc as plsc

_NC = 2
_NS = 16
_L = 16

_CHUNK = 128


@functools.lru_cache(maxsize=None)
def _make_sc_spmm(N, D, R, EPTP, NPAD):
    assert R % _NC == 0 and NPAD % (8 * _NS) == 0 and EPTP % _CHUNK == 0
    rpc = R // _NC
    n_chunks = EPTP // _CHUNK
    rpt = NPAD // _NS
    mesh = plsc.VectorSubcoreMesh(core_axis_name="c", subcore_axis_name="s")

    @functools.partial(
        pl.kernel,
        out_type=jax.ShapeDtypeStruct((R * NPAD, D), jnp.float32),
        mesh=mesh,
        scratch_types=[
            pltpu.VMEM((_CHUNK,), jnp.int32),
            pltpu.VMEM((_CHUNK,), jnp.int32),
            pltpu.VMEM((_CHUNK,), jnp.float32),
            pltpu.VMEM((_CHUNK, D), jnp.float32),
            pltpu.VMEM_SHARED((NPAD, D), jnp.float32),
            pltpu.SemaphoreType.DMA,
        ],
    )
    def spmm(x_hbm, src_hbm, dst_hbm, w_hbm, out_hbm, src_v, dst_v, w_v,
             rows_v, acc, sem):
        cid = lax.axis_index("c")
        sid = lax.axis_index("s")
        row0 = sid * rpt

        for rr in range(rpc):
            r = cid * rpc + rr

            def _zb(i, carry):
                z = jnp.zeros((_L,), jnp.float32)
                for j in range(D // _L):
                    rows_v[i, pl.ds(j * _L, _L)] = z
                return carry
            lax.fori_loop(0, _CHUNK, _zb, 0)

            done = 0
            while done < rpt:
                nrows = min(_CHUNK, rpt - done)
                pltpu.sync_copy(rows_v.at[pl.ds(0, nrows)],
                                acc.at[pl.ds(row0 + done, nrows)])
                done += nrows
            plsc.subcore_barrier()

            seg = (r * _NS + sid) * n_chunks

            pltpu.sync_copy(src_hbm.at[seg], src_v)

            def _chunk(k, carry):
                desc = pltpu.async_copy(x_hbm.at[src_v], rows_v, sem)
                pltpu.sync_copy(dst_hbm.at[seg + k], dst_v)
                pltpu.sync_copy(w_hbm.at[seg + k], w_v)
                desc.wait()

                def _scale(c16, c2):
                    wvec = w_v[pl.ds(c16 * _L, _L)]
                    for e16 in range(_L):
                        w = wvec[e16]
                        e = c16 * _L + e16
                        for j in range(D // _L):
                            sl = pl.ds(j * _L, _L)
                            rows_v[e, sl] = rows_v[e, sl] * w
                    return c2
                lax.fori_loop(0, _CHUNK // _L, _scale, 0)

                desc_s = pltpu.async_copy(rows_v, acc.at[dst_v], sem,
                                          add=True)
                kn = jnp.minimum(k + 1, n_chunks - 1)
                pltpu.sync_copy(src_hbm.at[seg + kn], src_v)
                desc_s.wait()
                return carry
            lax.fori_loop(0, n_chunks, _chunk, 0)
            plsc.subcore_barrier()

            done = 0
            while done < rpt:
                nrows = min(_CHUNK, rpt - done)
                pltpu.sync_copy(acc.at[pl.ds(row0 + done, nrows)],
                                out_hbm.at[pl.ds(r * NPAD + row0 + done,
                                                 nrows)])
                done += nrows

    return spmm


@functools.lru_cache(maxsize=None)
def _make_dense(N, D, R, DOUT, B):
    assert N % B == 0
    grid = (N // B,)

    def body(s_ref, w_ref, sw_ref, b_ref, g_ref, be_ref, o_ref):
        ssum = s_ref[0]
        for r in range(1, R):
            ssum = ssum + s_ref[r]
        mu = jnp.mean(ssum, axis=-1, keepdims=True)
        d = ssum - mu
        var = jnp.mean(d * d, axis=-1, keepdims=True)
        sn = d * lax.rsqrt(var + 1e-6) * g_ref[...] + be_ref[...]
        acc = jnp.dot(sn, sw_ref[...], preferred_element_type=jnp.float32)
        for r in range(R):
            acc = acc + jnp.dot(s_ref[r], w_ref[r],
                                preferred_element_type=jnp.float32)
        o_ref[...] = acc * 0.5 + b_ref[...]

    return pl.pallas_call(
        body,
        grid=grid,
        in_specs=[
            pl.BlockSpec((R, B, D), lambda i: (0, i, 0)),
            pl.BlockSpec((R, D, DOUT), lambda i: (0, 0, 0)),
            pl.BlockSpec((D, DOUT), lambda i: (0, 0)),
            pl.BlockSpec((1, DOUT), lambda i: (0, 0)),
            pl.BlockSpec((1, D), lambda i: (0, 0)),
            pl.BlockSpec((1, D), lambda i: (0, 0)),
        ],
        out_specs=pl.BlockSpec((B, DOUT), lambda i: (i, 0)),
        out_shape=jax.ShapeDtypeStruct((N, DOUT), jnp.float32),
    )


def kernel(x, edge_index, edge_weight, weight, share_weight, bias,
           ln_gamma, ln_beta):
    N, D = x.shape
    R, _, E = edge_index.shape
    DOUT = weight.shape[1]

    ept = E // _NS
    n_chunks = -(-ept // _CHUNK)
    eptp = n_chunks * _CHUNK
    pad = eptp - ept

    src = edge_index[:, 0, :].reshape(R, _NS, ept)
    dst = edge_index[:, 1, :].reshape(R, _NS, ept)
    ew = edge_weight.reshape(R, _NS, ept)
    if pad:
        src = jnp.pad(src, ((0, 0), (0, 0), (0, pad)))
        dst = jnp.pad(dst, ((0, 0), (0, 0), (0, pad)))
        ew = jnp.pad(ew, ((0, 0), (0, 0), (0, pad)))
    srcc = src.reshape(R * _NS * n_chunks, _CHUNK)
    dstc = dst.reshape(R * _NS * n_chunks, _CHUNK)
    wchunks = ew.reshape(R * _NS * n_chunks, _CHUNK)

    npad = -(-N // (8 * _NS)) * (8 * _NS)
    supports = _make_sc_spmm(N, D, R, eptp, npad)(x, srcc, dstc, wchunks)
    s = supports.reshape(R, npad, D)

    dense = _make_dense(N, D, R, DOUT, B=1000)
    return dense(
        s,
        weight.reshape(R, D, DOUT),
        share_weight,
        bias.reshape(1, DOUT),
        ln_gamma.reshape(1, D),
        ln_beta.reshape(1, D),
    )
